# Initial kernel scaffold; baseline (speedup 1.0000x reference)
import jax
import jax.numpy as jnp
from jax.experimental import pallas as pl


def kernel(inputs, edge_index, W1, b1, W2, b2):
    raise NotImplementedError("write your pallas kernel here")


# speedup vs baseline: 9.8287x; 9.8287x over previous
import functools

import jax
import jax.numpy as jnp
from jax import lax
from jax.experimental import pallas as pl
from jax.experimental.pallas import tpu as pltpu
from jax.experimental.pallas import tpu_sc as plsc

_N = 10000
_D = 128
_DH = _D // 2
_E = 320000
_NC = 2
_NS = 16
_NW = _NC * _NS
_CHUNK = 80
_NCHUNK_DEG = _E // _NW // _CHUNK
_NCHUNK_AGG = _E // _NS // _CHUNK
_NPAD = 10240
_ROWS_PER_TILE = _NPAD // _NS
_ZROWS = 128
_NB = 1000


def _sc_degree(dst3, ones_h, zdeg_h):
    mesh = plsc.VectorSubcoreMesh(core_axis_name="c", subcore_axis_name="s")

    @functools.partial(
        pl.kernel,
        out_type=jax.ShapeDtypeStruct((_NC, _NPAD), jnp.float32),
        mesh=mesh,
        compiler_params=pltpu.CompilerParams(use_tc_tiling_on_sc=False),
        scratch_types=[
            pltpu.VMEM((_NCHUNK_DEG, _CHUNK), jnp.int32),
            pltpu.VMEM((_CHUNK,), jnp.float32),
            pltpu.VMEM_SHARED((_NPAD,), jnp.float32),
        ],
    )
    def deg_kernel(dst_h, ones_hbm, z_hbm, out_h, dst_v, ones_v, acc):
        c = lax.axis_index("c")
        s = lax.axis_index("s")
        wid = c * _NS + s
        pltpu.sync_copy(dst_h.at[wid], dst_v)
        pltpu.sync_copy(ones_hbm, ones_v)
        sl = pl.ds(s * (_NPAD // _NS), _NPAD // _NS)
        pltpu.sync_copy(z_hbm, acc.at[sl])
        plsc.subcore_barrier()

        def body(j, carry):
            pltpu.sync_copy(ones_v, acc.at[dst_v.at[j]], add=True)
            return carry

        lax.fori_loop(0, _NCHUNK_DEG, body, 0)
        plsc.subcore_barrier()
        pltpu.sync_copy(acc.at[sl], out_h.at[c].at[sl])

    return deg_kernel(dst3, ones_h, zdeg_h)


def _sc_aggregate(y, src3, dst3, zrows_h):
    mesh = plsc.VectorSubcoreMesh(core_axis_name="c", subcore_axis_name="s")

    @functools.partial(
        pl.kernel,
        out_type=jax.ShapeDtypeStruct((_NC, _NPAD, _DH), jnp.float32),
        mesh=mesh,
        compiler_params=pltpu.CompilerParams(use_tc_tiling_on_sc=False),
        scratch_types=[
            pltpu.VMEM((_NCHUNK_AGG, _CHUNK), jnp.int32),
            pltpu.VMEM((_NCHUNK_AGG, _CHUNK), jnp.int32),
            pltpu.VMEM((2, _CHUNK, _DH), jnp.float32),
            pltpu.VMEM((_ZROWS, _DH), jnp.float32),
            pltpu.VMEM_SHARED((_NPAD, _DH), jnp.float32),
            pltpu.SemaphoreType.DMA,
            pltpu.SemaphoreType.DMA,
        ],
    )
    def agg_kernel(y_h, src_h, dst_h, z_h, out_h,
                   src_v, dst_v, rows_v, zrow_v, acc, sem0, sem1):
        c = lax.axis_index("c")
        s = lax.axis_index("s")
        pltpu.sync_copy(src_h.at[s], src_v)
        pltpu.sync_copy(dst_h.at[s], dst_v)
        pltpu.sync_copy(z_h, zrow_v)
        for i in range(_ROWS_PER_TILE // _ZROWS):
            pltpu.sync_copy(
                zrow_v, acc.at[pl.ds(s * _ROWS_PER_TILE + i * _ZROWS, _ZROWS)])
        plsc.subcore_barrier()
        yc = y_h.at[c]

        def body(g, carry):
            j0 = g * 2
            j1 = j0 + 1
            cp0 = pltpu.async_copy(yc.at[src_v.at[j0]], rows_v.at[0], sem0)
            cp1 = pltpu.async_copy(yc.at[src_v.at[j1]], rows_v.at[1], sem1)
            cp0.wait()
            pltpu.sync_copy(rows_v.at[0], acc.at[dst_v.at[j0]], add=True)
            cp1.wait()
            pltpu.sync_copy(rows_v.at[1], acc.at[dst_v.at[j1]], add=True)
            return carry

        lax.fori_loop(0, _NCHUNK_AGG // 2, body, 0)
        plsc.subcore_barrier()
        rsl = pl.ds(s * _ROWS_PER_TILE, _ROWS_PER_TILE)
        pltpu.sync_copy(acc.at[rsl], out_h.at[c].at[rsl])

    return agg_kernel(y, src3, dst3, zrows_h)


def _norm_of(d0_ref, d1_ref):
    return lax.rsqrt(jnp.maximum(d0_ref[...] + d1_ref[...], 1.0))


def _tc1_body(d0_ref, d1_ref, x_ref, y_ref):
    yn = x_ref[...] * _norm_of(d0_ref, d1_ref)
    y_ref[0] = yn[:, :_DH]
    y_ref[1] = yn[:, _DH:]


def _tc1(d0, d1, x):
    return pl.pallas_call(
        _tc1_body,
        out_shape=jax.ShapeDtypeStruct((2, _N, _DH), jnp.float32),
        grid=(_N // _NB,),
        in_specs=[
            pl.BlockSpec((_NB, 1), lambda i: (i, 0)),
            pl.BlockSpec((_NB, 1), lambda i: (i, 0)),
            pl.BlockSpec((_NB, _D), lambda i: (i, 0)),
        ],
        out_specs=pl.BlockSpec((2, _NB, _DH), lambda i: (0, i, 0)),
    )(d0, d1, x)


def _tc2_body(d0_ref, d1_ref, p_ref, x_ref, w_ref, b_ref, y_ref):
    norm = _norm_of(d0_ref, d1_ref)
    agg = jnp.concatenate([p_ref[0], p_ref[1]], axis=1) * norm
    h = jnp.dot(agg, w_ref[...], preferred_element_type=jnp.float32)
    h = jnp.maximum(h + b_ref[...] + x_ref[...], 0.0)
    hn = h * norm
    y_ref[0] = hn[:, :_DH]
    y_ref[1] = hn[:, _DH:]


def _tc2(d0, d1, p, x, W, b):
    return pl.pallas_call(
        _tc2_body,
        out_shape=jax.ShapeDtypeStruct((2, _N, _DH), jnp.float32),
        grid=(_N // _NB,),
        in_specs=[
            pl.BlockSpec((_NB, 1), lambda i: (i, 0)),
            pl.BlockSpec((_NB, 1), lambda i: (i, 0)),
            pl.BlockSpec((2, _NB, _DH), lambda i: (0, i, 0)),
            pl.BlockSpec((_NB, _D), lambda i: (i, 0)),
            pl.BlockSpec((_D, _D), lambda i: (0, 0)),
            pl.BlockSpec((1, _D), lambda i: (0, 0)),
        ],
        out_specs=pl.BlockSpec((2, _NB, _DH), lambda i: (0, i, 0)),
    )(d0, d1, p, x, W, b)


def _tc3_body(d0_ref, d1_ref, p_ref, w_ref, b_ref, y_ref):
    norm = _norm_of(d0_ref, d1_ref)
    agg = jnp.concatenate([p_ref[0], p_ref[1]], axis=1) * norm
    h = jnp.dot(agg, w_ref[...], preferred_element_type=jnp.float32)
    y_ref[...] = h + b_ref[...]


def _tc3(d0, d1, p, W, b):
    return pl.pallas_call(
        _tc3_body,
        out_shape=jax.ShapeDtypeStruct((_N, _D), jnp.float32),
        grid=(_N // _NB,),
        in_specs=[
            pl.BlockSpec((_NB, 1), lambda i: (i, 0)),
            pl.BlockSpec((_NB, 1), lambda i: (i, 0)),
            pl.BlockSpec((2, _NB, _DH), lambda i: (0, i, 0)),
            pl.BlockSpec((_D, _D), lambda i: (0, 0)),
            pl.BlockSpec((1, _D), lambda i: (0, 0)),
        ],
        out_specs=pl.BlockSpec((_NB, _D), lambda i: (i, 0)),
    )(d0, d1, p, W, b)


def kernel(inputs, edge_index, W1, b1, W2, b2):
    x = inputs
    src = edge_index[0]
    dst = edge_index[1]
    src_deg3 = dst.reshape(_NW, _NCHUNK_DEG, _CHUNK)
    src3 = src.reshape(_NS, _NCHUNK_AGG, _CHUNK)
    dst3 = dst.reshape(_NS, _NCHUNK_AGG, _CHUNK)
    ones_h = jnp.ones((_CHUNK,), jnp.float32)
    zdeg_h = jnp.zeros((_NPAD // _NS,), jnp.float32)
    zrows_h = jnp.zeros((_ZROWS, _DH), jnp.float32)

    degp = _sc_degree(src_deg3, ones_h, zdeg_h)
    d0 = degp[0, :_N].reshape(_N, 1)
    d1 = degp[1, :_N].reshape(_N, 1)

    y1 = _tc1(d0, d1, x)
    p = _sc_aggregate(y1, src3, dst3, zrows_h)
    y2 = _tc2(d0, d1, p[:, :_N], x, W1, b1.reshape(1, _D))
    q = _sc_aggregate(y2, src3, dst3, zrows_h)
    out = _tc3(d0, d1, q[:, :_N], W2, b2.reshape(1, _D))
    return out


# speedup vs baseline: 14.2542x; 1.4503x over previous
import functools

import jax
import jax.numpy as jnp
from jax import lax
from jax.experimental import pallas as pl
from jax.experimental.pallas import tpu as pltpu
from jax.experimental.pallas import tpu_sc as plsc

_N = 10000
_D = 128
_DH = _D // 2
_E = 320000
_NC = 2
_NS = 16
_NW = _NC * _NS
_CHUNK = 80
_NCHUNK_DEG = _E // _NW // _CHUNK
_ACHUNK = 125
_NCHUNK_AGG = _E // _NS // _ACHUNK
_NBUF = 4
_NGROUP = _NCHUNK_AGG // _NBUF
_NPAD = 10240
_ROWS_PER_TILE = _NPAD // _NS
_ZROWS = 128
_NB = 1000


def _sc_degree(dst3, ones_h, zdeg_h):
    mesh = plsc.VectorSubcoreMesh(core_axis_name="c", subcore_axis_name="s")

    @functools.partial(
        pl.kernel,
        out_type=jax.ShapeDtypeStruct((_NC, _NPAD), jnp.float32),
        mesh=mesh,
        compiler_params=pltpu.CompilerParams(use_tc_tiling_on_sc=False),
        scratch_types=[
            pltpu.VMEM((_NCHUNK_DEG, _CHUNK), jnp.int32),
            pltpu.VMEM((_CHUNK,), jnp.float32),
            pltpu.VMEM_SHARED((_NPAD,), jnp.float32),
        ],
    )
    def deg_kernel(dst_h, ones_hbm, z_hbm, out_h, dst_v, ones_v, acc):
        c = lax.axis_index("c")
        s = lax.axis_index("s")
        wid = c * _NS + s
        pltpu.sync_copy(dst_h.at[wid], dst_v)
        pltpu.sync_copy(ones_hbm, ones_v)
        sl = pl.ds(s * (_NPAD // _NS), _NPAD // _NS)
        pltpu.sync_copy(z_hbm, acc.at[sl])
        plsc.subcore_barrier()

        def body(j, carry):
            pltpu.sync_copy(ones_v, acc.at[dst_v.at[j]], add=True)
            return carry

        lax.fori_loop(0, _NCHUNK_DEG, body, 0)
        plsc.subcore_barrier()
        pltpu.sync_copy(acc.at[sl], out_h.at[c].at[sl])

    return deg_kernel(dst3, ones_h, zdeg_h)


def _sc_aggregate(y, src3, dst3, zrows_h):
    mesh = plsc.VectorSubcoreMesh(core_axis_name="c", subcore_axis_name="s")

    @functools.partial(
        pl.kernel,
        out_type=jax.ShapeDtypeStruct((_NC, _NPAD, _DH), jnp.float32),
        mesh=mesh,
        compiler_params=pltpu.CompilerParams(use_tc_tiling_on_sc=False),
        scratch_types=[
            pltpu.VMEM((_NCHUNK_AGG, _ACHUNK), jnp.int32),
            pltpu.VMEM((_NCHUNK_AGG, _ACHUNK), jnp.int32),
            pltpu.VMEM((_NBUF, _ACHUNK, _DH), jnp.float32),
            pltpu.VMEM((_ZROWS, _DH), jnp.float32),
            pltpu.VMEM_SHARED((_NPAD, _DH), jnp.float32),
            [pltpu.SemaphoreType.DMA] * _NBUF,
            [pltpu.SemaphoreType.DMA] * _NBUF,
        ],
    )
    def agg_kernel(y_h, src_h, dst_h, z_h, out_h,
                   src_v, dst_v, rows_v, zrow_v, acc, sem_g, sem_s):
        c = lax.axis_index("c")
        s = lax.axis_index("s")
        pltpu.sync_copy(src_h.at[s], src_v)
        pltpu.sync_copy(dst_h.at[s], dst_v)
        pltpu.sync_copy(z_h, zrow_v)
        for i in range(_ROWS_PER_TILE // _ZROWS):
            pltpu.sync_copy(
                zrow_v, acc.at[pl.ds(s * _ROWS_PER_TILE + i * _ZROWS, _ZROWS)])
        plsc.subcore_barrier()
        yc = y_h.at[c]

        for b in range(_NBUF):
            pltpu.async_copy(yc.at[src_v.at[b]], rows_v.at[b], sem_g[b])

        def group(g, carry):
            base = g * _NBUF
            for b in range(_NBUF):
                j = base + b
                pltpu.make_async_copy(
                    yc.at[src_v.at[j]], rows_v.at[b], sem_g[b]).wait()
                pltpu.async_copy(
                    rows_v.at[b], acc.at[dst_v.at[j]], sem_s[b], add=True)
            for b in range(_NBUF):
                j = base + b
                pltpu.make_async_copy(
                    rows_v.at[b], acc.at[dst_v.at[j]], sem_s[b]).wait()

                @pl.when(g < _NGROUP - 1)
                def _():
                    pltpu.async_copy(
                        yc.at[src_v.at[base + _NBUF + b]], rows_v.at[b], sem_g[b])
            return carry

        lax.fori_loop(0, _NGROUP, group, 0)
        plsc.subcore_barrier()
        rsl = pl.ds(s * _ROWS_PER_TILE, _ROWS_PER_TILE)
        pltpu.sync_copy(acc.at[rsl], out_h.at[c].at[rsl])

    return agg_kernel(y, src3, dst3, zrows_h)


def _norm_of(d0_ref, d1_ref):
    return lax.rsqrt(jnp.maximum(d0_ref[...] + d1_ref[...], 1.0))


def _tc1_body(d0_ref, d1_ref, x_ref, y_ref):
    yn = x_ref[...] * _norm_of(d0_ref, d1_ref)
    y_ref[0] = yn[:, :_DH]
    y_ref[1] = yn[:, _DH:]


def _tc1(d0, d1, x):
    return pl.pallas_call(
        _tc1_body,
        out_shape=jax.ShapeDtypeStruct((2, _N, _DH), jnp.float32),
        grid=(_N // _NB,),
        in_specs=[
            pl.BlockSpec((_NB, 1), lambda i: (i, 0)),
            pl.BlockSpec((_NB, 1), lambda i: (i, 0)),
            pl.BlockSpec((_NB, _D), lambda i: (i, 0)),
        ],
        out_specs=pl.BlockSpec((2, _NB, _DH), lambda i: (0, i, 0)),
    )(d0, d1, x)


def _tc2_body(d0_ref, d1_ref, p_ref, x_ref, w_ref, b_ref, y_ref):
    norm = _norm_of(d0_ref, d1_ref)
    agg = jnp.concatenate([p_ref[0], p_ref[1]], axis=1) * norm
    h = jnp.dot(agg, w_ref[...], preferred_element_type=jnp.float32)
    h = jnp.maximum(h + b_ref[...] + x_ref[...], 0.0)
    hn = h * norm
    y_ref[0] = hn[:, :_DH]
    y_ref[1] = hn[:, _DH:]


def _tc2(d0, d1, p, x, W, b):
    return pl.pallas_call(
        _tc2_body,
        out_shape=jax.ShapeDtypeStruct((2, _N, _DH), jnp.float32),
        grid=(_N // _NB,),
        in_specs=[
            pl.BlockSpec((_NB, 1), lambda i: (i, 0)),
            pl.BlockSpec((_NB, 1), lambda i: (i, 0)),
            pl.BlockSpec((2, _NB, _DH), lambda i: (0, i, 0)),
            pl.BlockSpec((_NB, _D), lambda i: (i, 0)),
            pl.BlockSpec((_D, _D), lambda i: (0, 0)),
            pl.BlockSpec((1, _D), lambda i: (0, 0)),
        ],
        out_specs=pl.BlockSpec((2, _NB, _DH), lambda i: (0, i, 0)),
    )(d0, d1, p, x, W, b)


def _tc3_body(d0_ref, d1_ref, p_ref, w_ref, b_ref, y_ref):
    norm = _norm_of(d0_ref, d1_ref)
    agg = jnp.concatenate([p_ref[0], p_ref[1]], axis=1) * norm
    h = jnp.dot(agg, w_ref[...], preferred_element_type=jnp.float32)
    y_ref[...] = h + b_ref[...]


def _tc3(d0, d1, p, W, b):
    return pl.pallas_call(
        _tc3_body,
        out_shape=jax.ShapeDtypeStruct((_N, _D), jnp.float32),
        grid=(_N // _NB,),
        in_specs=[
            pl.BlockSpec((_NB, 1), lambda i: (i, 0)),
            pl.BlockSpec((_NB, 1), lambda i: (i, 0)),
            pl.BlockSpec((2, _NB, _DH), lambda i: (0, i, 0)),
            pl.BlockSpec((_D, _D), lambda i: (0, 0)),
            pl.BlockSpec((1, _D), lambda i: (0, 0)),
        ],
        out_specs=pl.BlockSpec((_NB, _D), lambda i: (i, 0)),
    )(d0, d1, p, W, b)


def kernel(inputs, edge_index, W1, b1, W2, b2):
    x = inputs
    src = edge_index[0]
    dst = edge_index[1]
    src_deg3 = dst.reshape(_NW, _NCHUNK_DEG, _CHUNK)
    src3 = src.reshape(_NS, _NCHUNK_AGG, _ACHUNK)
    dst3 = dst.reshape(_NS, _NCHUNK_AGG, _ACHUNK)
    ones_h = jnp.ones((_CHUNK,), jnp.float32)
    zdeg_h = jnp.zeros((_NPAD // _NS,), jnp.float32)
    zrows_h = jnp.zeros((_ZROWS, _DH), jnp.float32)

    degp = _sc_degree(src_deg3, ones_h, zdeg_h)
    d0 = degp[0, :_N].reshape(_N, 1)
    d1 = degp[1, :_N].reshape(_N, 1)

    y1 = _tc1(d0, d1, x)
    p = _sc_aggregate(y1, src3, dst3, zrows_h)
    y2 = _tc2(d0, d1, p[:, :_N], x, W1, b1.reshape(1, _D))
    q = _sc_aggregate(y2, src3, dst3, zrows_h)
    out = _tc3(d0, d1, q[:, :_N], W2, b2.reshape(1, _D))
    return out


# speedup vs baseline: 16.8497x; 1.1821x over previous
import functools

import jax
import jax.numpy as jnp
from jax import lax
from jax.experimental import pallas as pl
from jax.experimental.pallas import tpu as pltpu
from jax.experimental.pallas import tpu_sc as plsc

_N = 10000
_D = 128
_DH = _D // 2
_E = 320000
_NC = 2
_NS = 16
_NW = _NC * _NS
_CHUNK = 80
_NCHUNK_DEG = _E // _NW // _CHUNK
_ACHUNK = 125
_NCHUNK_AGG = _E // _NS // _ACHUNK
_NBUF = 4
_NGROUP = _NCHUNK_AGG // _NBUF
_NPAD = 10240
_ROWS_PER_TILE = _NPAD // _NS
_ZROWS = 128
_NB = 1000


def _sc_degree(dst3, ones_h, zdeg_h):
    mesh = plsc.VectorSubcoreMesh(core_axis_name="c", subcore_axis_name="s")

    @functools.partial(
        pl.kernel,
        out_type=jax.ShapeDtypeStruct((_NC, _NPAD), jnp.float32),
        mesh=mesh,
        compiler_params=pltpu.CompilerParams(use_tc_tiling_on_sc=False),
        scratch_types=[
            pltpu.VMEM((_NCHUNK_DEG, _CHUNK), jnp.int32),
            pltpu.VMEM((_CHUNK,), jnp.float32),
            pltpu.VMEM_SHARED((_NPAD,), jnp.float32),
        ],
    )
    def deg_kernel(dst_h, ones_hbm, z_hbm, out_h, dst_v, ones_v, acc):
        c = lax.axis_index("c")
        s = lax.axis_index("s")
        wid = c * _NS + s
        pltpu.sync_copy(dst_h.at[wid], dst_v)
        pltpu.sync_copy(ones_hbm, ones_v)
        sl = pl.ds(s * (_NPAD // _NS), _NPAD // _NS)
        pltpu.sync_copy(z_hbm, acc.at[sl])
        plsc.subcore_barrier()

        def body(j, carry):
            pltpu.sync_copy(ones_v, acc.at[dst_v.at[j]], add=True)
            return carry

        lax.fori_loop(0, _NCHUNK_DEG, body, 0)
        plsc.subcore_barrier()
        pltpu.sync_copy(acc.at[sl], out_h.at[c].at[sl])

    return deg_kernel(dst3, ones_h, zdeg_h)


def _sc_aggregate(y2d, idx4, dst3, zrows_h):
    mesh = plsc.VectorSubcoreMesh(core_axis_name="c", subcore_axis_name="s")

    @functools.partial(
        pl.kernel,
        out_type=jax.ShapeDtypeStruct((_NPAD, _D), jnp.float32),
        mesh=mesh,
        compiler_params=pltpu.CompilerParams(use_tc_tiling_on_sc=False),
        scratch_types=[
            pltpu.VMEM((_NCHUNK_AGG, _ACHUNK), jnp.int32),
            pltpu.VMEM((_NCHUNK_AGG, _ACHUNK), jnp.int32),
            pltpu.VMEM((_NBUF, _ACHUNK, _DH), jnp.float32),
            pltpu.VMEM((_ZROWS, _DH), jnp.float32),
            pltpu.VMEM_SHARED((_NPAD, _DH), jnp.float32),
            [pltpu.SemaphoreType.DMA] * _NBUF,
            [pltpu.SemaphoreType.DMA] * _NBUF,
        ],
    )
    def agg_kernel(y_h, src_h, dst_h, z_h, out_h,
                   src_v, dst_v, rows_v, zrow_v, acc, sem_g, sem_s):
        c = lax.axis_index("c")
        s = lax.axis_index("s")
        pltpu.sync_copy(src_h.at[c].at[s], src_v)
        pltpu.sync_copy(dst_h.at[s], dst_v)
        pltpu.sync_copy(z_h, zrow_v)
        for i in range(_ROWS_PER_TILE // _ZROWS):
            pltpu.sync_copy(
                zrow_v, acc.at[pl.ds(s * _ROWS_PER_TILE + i * _ZROWS, _ZROWS)])
        plsc.subcore_barrier()
        yc = y_h

        for b in range(_NBUF):
            pltpu.async_copy(yc.at[src_v.at[b]], rows_v.at[b], sem_g[b])

        def group(g, carry):
            base = g * _NBUF
            for b in range(_NBUF):
                j = base + b
                pltpu.make_async_copy(
                    yc.at[src_v.at[j]], rows_v.at[b], sem_g[b]).wait()
                pltpu.async_copy(
                    rows_v.at[b], acc.at[dst_v.at[j]], sem_s[b], add=True)
            for b in range(_NBUF):
                j = base + b
                pltpu.make_async_copy(
                    rows_v.at[b], acc.at[dst_v.at[j]], sem_s[b]).wait()

                @pl.when(g < _NGROUP - 1)
                def _():
                    pltpu.async_copy(
                        yc.at[src_v.at[base + _NBUF + b]], rows_v.at[b], sem_g[b])
            return carry

        lax.fori_loop(0, _NGROUP, group, 0)
        plsc.subcore_barrier()
        rsl = pl.ds(s * _ROWS_PER_TILE, _ROWS_PER_TILE)
        pltpu.sync_copy(acc.at[rsl], out_h.at[rsl, pl.ds(c * _DH, _DH)])

    return agg_kernel(y2d, idx4, dst3, zrows_h)


def _norm_of(d0_ref, d1_ref):
    return lax.rsqrt(jnp.maximum(d0_ref[...] + d1_ref[...], 1.0))


def _tc1_body(d0_ref, d1_ref, x_ref, y_ref):
    y_ref[...] = x_ref[...] * _norm_of(d0_ref, d1_ref)


def _tc1(d0, d1, x):
    return pl.pallas_call(
        _tc1_body,
        out_shape=jax.ShapeDtypeStruct((_N, _D), jnp.float32),
        grid=(_N // _NB,),
        in_specs=[
            pl.BlockSpec((_NB, 1), lambda i: (i, 0)),
            pl.BlockSpec((_NB, 1), lambda i: (i, 0)),
            pl.BlockSpec((_NB, _D), lambda i: (i, 0)),
        ],
        out_specs=pl.BlockSpec((_NB, _D), lambda i: (i, 0)),
    )(d0, d1, x)


def _tc2_body(d0_ref, d1_ref, p_ref, x_ref, w_ref, b_ref, y_ref):
    norm = _norm_of(d0_ref, d1_ref)
    agg = p_ref[...] * norm
    h = jnp.dot(agg, w_ref[...], preferred_element_type=jnp.float32)
    h = jnp.maximum(h + b_ref[...] + x_ref[...], 0.0)
    y_ref[...] = h * norm


def _tc2(d0, d1, p, x, W, b):
    return pl.pallas_call(
        _tc2_body,
        out_shape=jax.ShapeDtypeStruct((_N, _D), jnp.float32),
        grid=(_N // _NB,),
        in_specs=[
            pl.BlockSpec((_NB, 1), lambda i: (i, 0)),
            pl.BlockSpec((_NB, 1), lambda i: (i, 0)),
            pl.BlockSpec((_NB, _D), lambda i: (i, 0)),
            pl.BlockSpec((_NB, _D), lambda i: (i, 0)),
            pl.BlockSpec((_D, _D), lambda i: (0, 0)),
            pl.BlockSpec((1, _D), lambda i: (0, 0)),
        ],
        out_specs=pl.BlockSpec((_NB, _D), lambda i: (i, 0)),
    )(d0, d1, p, x, W, b)


def _tc3_body(d0_ref, d1_ref, p_ref, w_ref, b_ref, y_ref):
    norm = _norm_of(d0_ref, d1_ref)
    agg = p_ref[...] * norm
    h = jnp.dot(agg, w_ref[...], preferred_element_type=jnp.float32)
    y_ref[...] = h + b_ref[...]


def _tc3(d0, d1, p, W, b):
    return pl.pallas_call(
        _tc3_body,
        out_shape=jax.ShapeDtypeStruct((_N, _D), jnp.float32),
        grid=(_N // _NB,),
        in_specs=[
            pl.BlockSpec((_NB, 1), lambda i: (i, 0)),
            pl.BlockSpec((_NB, 1), lambda i: (i, 0)),
            pl.BlockSpec((_NB, _D), lambda i: (i, 0)),
            pl.BlockSpec((_D, _D), lambda i: (0, 0)),
            pl.BlockSpec((1, _D), lambda i: (0, 0)),
        ],
        out_specs=pl.BlockSpec((_NB, _D), lambda i: (i, 0)),
    )(d0, d1, p, W, b)


def kernel(inputs, edge_index, W1, b1, W2, b2):
    x = inputs
    src = edge_index[0]
    dst = edge_index[1]
    src_deg3 = dst.reshape(_NW, _NCHUNK_DEG, _CHUNK)
    idx4 = (2 * src)[None, :] + jnp.arange(2, dtype=jnp.int32)[:, None]
    idx4 = idx4.reshape(_NC, _NS, _NCHUNK_AGG, _ACHUNK)
    dst3 = dst.reshape(_NS, _NCHUNK_AGG, _ACHUNK)
    ones_h = jnp.ones((_CHUNK,), jnp.float32)
    zdeg_h = jnp.zeros((_NPAD // _NS,), jnp.float32)
    zrows_h = jnp.zeros((_ZROWS, _DH), jnp.float32)

    degp = _sc_degree(src_deg3, ones_h, zdeg_h)
    d0 = degp[0, :_N].reshape(_N, 1)
    d1 = degp[1, :_N].reshape(_N, 1)

    y1 = _tc1(d0, d1, x)
    p = _sc_aggregate(y1.reshape(2 * _N, _DH), idx4, dst3, zrows_h)
    y2 = _tc2(d0, d1, p, x, W1, b1.reshape(1, _D))
    q = _sc_aggregate(y2.reshape(2 * _N, _DH), idx4, dst3, zrows_h)
    out = _tc3(d0, d1, q, W2, b2.reshape(1, _D))
    return out


# speedup vs baseline: 17.3955x; 1.0324x over previous
import functools

import jax
import jax.numpy as jnp
from jax import lax
from jax.experimental import pallas as pl
from jax.experimental.pallas import tpu as pltpu
from jax.experimental.pallas import tpu_sc as plsc

_N = 10000
_D = 128
_DH = _D // 2
_E = 320000
_NC = 2
_NS = 16
_NW = _NC * _NS
_ACHUNK = 80
_NCHUNK_AGG = _E // _NS // _ACHUNK
_NCHUNK_DEG = _NCHUNK_AGG // _NC
_NBUF = 5
_NGROUP = _NCHUNK_AGG // _NBUF
_NPAD = 10240
_ROWS_PER_TILE = _NPAD // _NS
_ZROWS = 128
_NB = 1000


def _sc_degree(edge4, ones_h, zdeg_h):
    mesh = plsc.VectorSubcoreMesh(core_axis_name="c", subcore_axis_name="s")

    @functools.partial(
        pl.kernel,
        out_type=jax.ShapeDtypeStruct((_NC, _NPAD), jnp.float32),
        mesh=mesh,
        compiler_params=pltpu.CompilerParams(use_tc_tiling_on_sc=False),
        scratch_types=[
            pltpu.VMEM((_NCHUNK_DEG, _ACHUNK), jnp.int32),
            pltpu.VMEM((_ACHUNK,), jnp.float32),
            pltpu.VMEM_SHARED((_NPAD,), jnp.float32),
        ],
    )
    def deg_kernel(edge_h, ones_hbm, z_hbm, out_h, dst_v, ones_v, acc):
        c = lax.axis_index("c")
        s = lax.axis_index("s")
        pltpu.sync_copy(
            edge_h.at[1].at[s].at[pl.ds(c * _NCHUNK_DEG, _NCHUNK_DEG)], dst_v)
        pltpu.sync_copy(ones_hbm, ones_v)
        sl = pl.ds(s * (_NPAD // _NS), _NPAD // _NS)
        pltpu.sync_copy(z_hbm, acc.at[sl])
        plsc.subcore_barrier()

        def body(j, carry):
            pltpu.sync_copy(ones_v, acc.at[dst_v.at[j]], add=True)
            return carry

        lax.fori_loop(0, _NCHUNK_DEG, body, 0)
        plsc.subcore_barrier()
        pltpu.sync_copy(acc.at[sl], out_h.at[c].at[sl])

    return deg_kernel(edge4, ones_h, zdeg_h)


def _sc_aggregate(y2d, edge4, zrows_h):
    mesh = plsc.VectorSubcoreMesh(core_axis_name="c", subcore_axis_name="s")

    @functools.partial(
        pl.kernel,
        out_type=jax.ShapeDtypeStruct((_NPAD, _D), jnp.float32),
        mesh=mesh,
        compiler_params=pltpu.CompilerParams(use_tc_tiling_on_sc=False),
        scratch_types=[
            pltpu.VMEM((_NCHUNK_AGG, _ACHUNK), jnp.int32),
            pltpu.VMEM((_NCHUNK_AGG, _ACHUNK), jnp.int32),
            pltpu.VMEM((_NBUF, _ACHUNK, _DH), jnp.float32),
            pltpu.VMEM((_ZROWS, _DH), jnp.float32),
            pltpu.VMEM_SHARED((_NPAD, _DH), jnp.float32),
            [pltpu.SemaphoreType.DMA] * _NBUF,
            [pltpu.SemaphoreType.DMA] * _NBUF,
        ],
    )
    def agg_kernel(y_h, edge_h, z_h, out_h,
                   src_v, dst_v, rows_v, zrow_v, acc, sem_g, sem_s):
        c = lax.axis_index("c")
        s = lax.axis_index("s")
        pltpu.sync_copy(edge_h.at[0].at[s], src_v)
        pltpu.sync_copy(edge_h.at[1].at[s], dst_v)
        two = jnp.int32(2)

        def transform(j, carry):
            for t in range(_ACHUNK // 16):
                sl = pl.ds(t * 16, 16)
                src_v[j, sl] = src_v[j, sl] * two + c
            return carry

        lax.fori_loop(0, _NCHUNK_AGG, transform, 0)
        pltpu.sync_copy(z_h, zrow_v)
        for i in range(_ROWS_PER_TILE // _ZROWS):
            pltpu.sync_copy(
                zrow_v, acc.at[pl.ds(s * _ROWS_PER_TILE + i * _ZROWS, _ZROWS)])
        plsc.subcore_barrier()
        yc = y_h

        for b in range(_NBUF):
            pltpu.async_copy(yc.at[src_v.at[b]], rows_v.at[b], sem_g[b])

        def group(g, carry):
            base = g * _NBUF
            for b in range(_NBUF):
                j = base + b
                pltpu.make_async_copy(
                    yc.at[src_v.at[j]], rows_v.at[b], sem_g[b]).wait()
                pltpu.async_copy(
                    rows_v.at[b], acc.at[dst_v.at[j]], sem_s[b], add=True)
            for b in range(_NBUF):
                j = base + b
                pltpu.make_async_copy(
                    rows_v.at[b], acc.at[dst_v.at[j]], sem_s[b]).wait()

                @pl.when(g < _NGROUP - 1)
                def _():
                    pltpu.async_copy(
                        yc.at[src_v.at[base + _NBUF + b]], rows_v.at[b], sem_g[b])
            return carry

        lax.fori_loop(0, _NGROUP, group, 0)
        plsc.subcore_barrier()
        rsl = pl.ds(s * _ROWS_PER_TILE, _ROWS_PER_TILE)
        pltpu.sync_copy(acc.at[rsl], out_h.at[rsl, pl.ds(c * _DH, _DH)])

    return agg_kernel(y2d, edge4, zrows_h)


def _norm_of(d0_ref, d1_ref):
    return lax.rsqrt(jnp.maximum(d0_ref[...] + d1_ref[...], 1.0))


def _tc1_body(d0_ref, d1_ref, x_ref, y_ref):
    y_ref[...] = x_ref[...] * _norm_of(d0_ref, d1_ref)


def _tc1(d0, d1, x):
    return pl.pallas_call(
        _tc1_body,
        out_shape=jax.ShapeDtypeStruct((_N, _D), jnp.float32),
        grid=(_N // _NB,),
        in_specs=[
            pl.BlockSpec((_NB, 1), lambda i: (i, 0)),
            pl.BlockSpec((_NB, 1), lambda i: (i, 0)),
            pl.BlockSpec((_NB, _D), lambda i: (i, 0)),
        ],
        out_specs=pl.BlockSpec((_NB, _D), lambda i: (i, 0)),
    )(d0, d1, x)


def _tc2_body(d0_ref, d1_ref, p_ref, x_ref, w_ref, b_ref, y_ref):
    norm = _norm_of(d0_ref, d1_ref)
    agg = p_ref[...] * norm
    h = jnp.dot(agg, w_ref[...], preferred_element_type=jnp.float32)
    h = jnp.maximum(h + b_ref[...] + x_ref[...], 0.0)
    y_ref[...] = h * norm


def _tc2(d0, d1, p, x, W, b):
    return pl.pallas_call(
        _tc2_body,
        out_shape=jax.ShapeDtypeStruct((_N, _D), jnp.float32),
        grid=(_N // _NB,),
        in_specs=[
            pl.BlockSpec((_NB, 1), lambda i: (i, 0)),
            pl.BlockSpec((_NB, 1), lambda i: (i, 0)),
            pl.BlockSpec((_NB, _D), lambda i: (i, 0)),
            pl.BlockSpec((_NB, _D), lambda i: (i, 0)),
            pl.BlockSpec((_D, _D), lambda i: (0, 0)),
            pl.BlockSpec((1, _D), lambda i: (0, 0)),
        ],
        out_specs=pl.BlockSpec((_NB, _D), lambda i: (i, 0)),
    )(d0, d1, p, x, W, b)


def _tc3_body(d0_ref, d1_ref, p_ref, w_ref, b_ref, y_ref):
    norm = _norm_of(d0_ref, d1_ref)
    agg = p_ref[...] * norm
    h = jnp.dot(agg, w_ref[...], preferred_element_type=jnp.float32)
    y_ref[...] = h + b_ref[...]


def _tc3(d0, d1, p, W, b):
    return pl.pallas_call(
        _tc3_body,
        out_shape=jax.ShapeDtypeStruct((_N, _D), jnp.float32),
        grid=(_N // _NB,),
        in_specs=[
            pl.BlockSpec((_NB, 1), lambda i: (i, 0)),
            pl.BlockSpec((_NB, 1), lambda i: (i, 0)),
            pl.BlockSpec((_NB, _D), lambda i: (i, 0)),
            pl.BlockSpec((_D, _D), lambda i: (0, 0)),
            pl.BlockSpec((1, _D), lambda i: (0, 0)),
        ],
        out_specs=pl.BlockSpec((_NB, _D), lambda i: (i, 0)),
    )(d0, d1, p, W, b)


def kernel(inputs, edge_index, W1, b1, W2, b2):
    x = inputs
    edge_agg = edge_index.reshape(2, _NS, _NCHUNK_AGG, _ACHUNK)
    ones_h = jnp.ones((_ACHUNK,), jnp.float32)
    zdeg_h = jnp.zeros((_NPAD // _NS,), jnp.float32)
    zrows_h = jnp.zeros((_ZROWS, _DH), jnp.float32)

    degp = _sc_degree(edge_agg, ones_h, zdeg_h)
    d0 = degp[0, :_N].reshape(_N, 1)
    d1 = degp[1, :_N].reshape(_N, 1)

    y1 = _tc1(d0, d1, x)
    p = _sc_aggregate(y1.reshape(_N * _NC, _DH), edge_agg, zrows_h)
    y2 = _tc2(d0, d1, p, x, W1, b1.reshape(1, _D))
    q = _sc_aggregate(y2.reshape(_N * _NC, _DH), edge_agg, zrows_h)
    out = _tc3(d0, d1, q, W2, b2.reshape(1, _D))
    return out


# speedup vs baseline: 17.7965x; 1.0231x over previous
import functools

import jax
import jax.numpy as jnp
from jax import lax
from jax.experimental import pallas as pl
from jax.experimental.pallas import tpu as pltpu
from jax.experimental.pallas import tpu_sc as plsc

_N = 10000
_D = 128
_DH = _D // 2
_E = 320000
_NC = 2
_NS = 16
_NW = _NC * _NS
_ACHUNK = 80
_NCHUNK_AGG = _E // _NS // _ACHUNK
_NCHUNK_DEG = _NCHUNK_AGG // _NC
_NBUF = 5
_NGROUP = _NCHUNK_AGG // _NBUF
_NPAD = 10240
_ROWS_PER_TILE = _NPAD // _NS
_ZROWS = 128
_NB = 2000


def _sc_degree(edge4, ones_h, zdeg_h):
    mesh = plsc.VectorSubcoreMesh(core_axis_name="c", subcore_axis_name="s")

    @functools.partial(
        pl.kernel,
        out_type=jax.ShapeDtypeStruct((_NC, _NPAD), jnp.float32),
        mesh=mesh,
        compiler_params=pltpu.CompilerParams(use_tc_tiling_on_sc=False),
        scratch_types=[
            pltpu.VMEM((_NCHUNK_DEG, _ACHUNK), jnp.int32),
            pltpu.VMEM((_ACHUNK,), jnp.float32),
            pltpu.VMEM_SHARED((_NPAD,), jnp.float32),
        ],
    )
    def deg_kernel(edge_h, ones_hbm, z_hbm, out_h, dst_v, ones_v, acc):
        c = lax.axis_index("c")
        s = lax.axis_index("s")
        pltpu.sync_copy(
            edge_h.at[1].at[s].at[pl.ds(c * _NCHUNK_DEG, _NCHUNK_DEG)], dst_v)
        pltpu.sync_copy(ones_hbm, ones_v)
        sl = pl.ds(s * (_NPAD // _NS), _NPAD // _NS)
        pltpu.sync_copy(z_hbm, acc.at[sl])
        plsc.subcore_barrier()

        def body(j, carry):
            pltpu.sync_copy(ones_v, acc.at[dst_v.at[j]], add=True)
            return carry

        lax.fori_loop(0, _NCHUNK_DEG, body, 0)
        plsc.subcore_barrier()
        pltpu.sync_copy(acc.at[sl], out_h.at[c].at[sl])

    return deg_kernel(edge4, ones_h, zdeg_h)


def _sc_aggregate(y2d, edge4, zrows_h):
    mesh = plsc.VectorSubcoreMesh(core_axis_name="c", subcore_axis_name="s")

    @functools.partial(
        pl.kernel,
        out_type=jax.ShapeDtypeStruct((_NPAD, _D), jnp.float32),
        mesh=mesh,
        compiler_params=pltpu.CompilerParams(use_tc_tiling_on_sc=False),
        scratch_types=[
            pltpu.VMEM((_NCHUNK_AGG, _ACHUNK), jnp.int32),
            pltpu.VMEM((_NCHUNK_AGG, _ACHUNK), jnp.int32),
            pltpu.VMEM((_NBUF, _ACHUNK, _DH), jnp.float32),
            pltpu.VMEM((_ZROWS, _DH), jnp.float32),
            pltpu.VMEM_SHARED((_NPAD, _DH), jnp.float32),
            [pltpu.SemaphoreType.DMA] * _NBUF,
            [pltpu.SemaphoreType.DMA] * _NBUF,
        ],
    )
    def agg_kernel(y_h, edge_h, z_h, out_h,
                   src_v, dst_v, rows_v, zrow_v, acc, sem_g, sem_s):
        c = lax.axis_index("c")
        s = lax.axis_index("s")
        pltpu.sync_copy(edge_h.at[0].at[s], src_v)
        pltpu.sync_copy(edge_h.at[1].at[s], dst_v)
        two = jnp.int32(2)

        def transform(j, carry):
            for t in range(_ACHUNK // 16):
                sl = pl.ds(t * 16, 16)
                src_v[j, sl] = src_v[j, sl] * two + c
            return carry

        lax.fori_loop(0, _NCHUNK_AGG, transform, 0)
        pltpu.sync_copy(z_h, zrow_v)
        for i in range(_ROWS_PER_TILE // _ZROWS):
            pltpu.sync_copy(
                zrow_v, acc.at[pl.ds(s * _ROWS_PER_TILE + i * _ZROWS, _ZROWS)])
        plsc.subcore_barrier()
        yc = y_h

        for b in range(_NBUF):
            pltpu.async_copy(yc.at[src_v.at[b]], rows_v.at[b], sem_g[b])

        def group(g, carry):
            base = g * _NBUF
            for b in range(_NBUF):
                j = base + b
                pltpu.make_async_copy(
                    yc.at[src_v.at[j]], rows_v.at[b], sem_g[b]).wait()
                pltpu.async_copy(
                    rows_v.at[b], acc.at[dst_v.at[j]], sem_s[b], add=True)
            for b in range(_NBUF):
                j = base + b
                pltpu.make_async_copy(
                    rows_v.at[b], acc.at[dst_v.at[j]], sem_s[b]).wait()

                @pl.when(g < _NGROUP - 1)
                def _():
                    pltpu.async_copy(
                        yc.at[src_v.at[base + _NBUF + b]], rows_v.at[b], sem_g[b])
            return carry

        lax.fori_loop(0, _NGROUP, group, 0)
        plsc.subcore_barrier()
        rsl = pl.ds(s * _ROWS_PER_TILE, _ROWS_PER_TILE)
        pltpu.sync_copy(acc.at[rsl], out_h.at[rsl, pl.ds(c * _DH, _DH)])

    return agg_kernel(y2d, edge4, zrows_h)


def _norm_of(d0_ref, d1_ref):
    return lax.rsqrt(jnp.maximum(d0_ref[...] + d1_ref[...], 1.0))


def _tc1_body(d0_ref, d1_ref, x_ref, y_ref):
    y_ref[...] = x_ref[...] * _norm_of(d0_ref, d1_ref)


def _tc1(d0, d1, x):
    return pl.pallas_call(
        _tc1_body,
        out_shape=jax.ShapeDtypeStruct((_N, _D), jnp.float32),
        grid=(_N // _NB,),
        in_specs=[
            pl.BlockSpec((_NB, 1), lambda i: (i, 0)),
            pl.BlockSpec((_NB, 1), lambda i: (i, 0)),
            pl.BlockSpec((_NB, _D), lambda i: (i, 0)),
        ],
        out_specs=pl.BlockSpec((_NB, _D), lambda i: (i, 0)),
    )(d0, d1, x)


def _tc2_body(d0_ref, d1_ref, p_ref, x_ref, w_ref, b_ref, y_ref):
    norm = _norm_of(d0_ref, d1_ref)
    agg = p_ref[...] * norm
    h = jnp.dot(agg, w_ref[...], preferred_element_type=jnp.float32)
    h = jnp.maximum(h + b_ref[...] + x_ref[...], 0.0)
    y_ref[...] = h * norm


def _tc2(d0, d1, p, x, W, b):
    return pl.pallas_call(
        _tc2_body,
        out_shape=jax.ShapeDtypeStruct((_N, _D), jnp.float32),
        grid=(_N // _NB,),
        in_specs=[
            pl.BlockSpec((_NB, 1), lambda i: (i, 0)),
            pl.BlockSpec((_NB, 1), lambda i: (i, 0)),
            pl.BlockSpec((_NB, _D), lambda i: (i, 0)),
            pl.BlockSpec((_NB, _D), lambda i: (i, 0)),
            pl.BlockSpec((_D, _D), lambda i: (0, 0)),
            pl.BlockSpec((1, _D), lambda i: (0, 0)),
        ],
        out_specs=pl.BlockSpec((_NB, _D), lambda i: (i, 0)),
    )(d0, d1, p, x, W, b)


def _tc3_body(d0_ref, d1_ref, p_ref, w_ref, b_ref, y_ref):
    norm = _norm_of(d0_ref, d1_ref)
    agg = p_ref[...] * norm
    h = jnp.dot(agg, w_ref[...], preferred_element_type=jnp.float32)
    y_ref[...] = h + b_ref[...]


def _tc3(d0, d1, p, W, b):
    return pl.pallas_call(
        _tc3_body,
        out_shape=jax.ShapeDtypeStruct((_N, _D), jnp.float32),
        grid=(_N // _NB,),
        in_specs=[
            pl.BlockSpec((_NB, 1), lambda i: (i, 0)),
            pl.BlockSpec((_NB, 1), lambda i: (i, 0)),
            pl.BlockSpec((_NB, _D), lambda i: (i, 0)),
            pl.BlockSpec((_D, _D), lambda i: (0, 0)),
            pl.BlockSpec((1, _D), lambda i: (0, 0)),
        ],
        out_specs=pl.BlockSpec((_NB, _D), lambda i: (i, 0)),
    )(d0, d1, p, W, b)


def kernel(inputs, edge_index, W1, b1, W2, b2):
    x = inputs
    edge_agg = edge_index.reshape(2, _NS, _NCHUNK_AGG, _ACHUNK)
    ones_h = jnp.ones((_ACHUNK,), jnp.float32)
    zdeg_h = jnp.zeros((_NPAD // _NS,), jnp.float32)
    zrows_h = jnp.zeros((_ZROWS, _DH), jnp.float32)

    degp = _sc_degree(edge_agg, ones_h, zdeg_h)
    d0 = degp[0, :_N].reshape(_N, 1)
    d1 = degp[1, :_N].reshape(_N, 1)

    y1 = _tc1(d0, d1, x)
    p = _sc_aggregate(y1.reshape(_N * _NC, _DH), edge_agg, zrows_h)
    y2 = _tc2(d0, d1, p, x, W1, b1.reshape(1, _D))
    q = _sc_aggregate(y2.reshape(_N * _NC, _DH), edge_agg, zrows_h)
    out = _tc3(d0, d1, q, W2, b2.reshape(1, _D))
    return out


# speedup vs baseline: 18.3150x; 1.0291x over previous
import functools

import jax
import jax.numpy as jnp
from jax import lax
from jax.experimental import pallas as pl
from jax.experimental.pallas import tpu as pltpu
from jax.experimental.pallas import tpu_sc as plsc

_N = 10000
_D = 128
_DH = _D // 2
_E = 320000
_NC = 2
_NS = 16
_NW = _NC * _NS
_ACHUNK = 80
_NCHUNK_AGG = _E // _NS // _ACHUNK
_NCHUNK_DEG = _NCHUNK_AGG // _NC
_NBUF = 5
_NGROUP = _NCHUNK_AGG // _NBUF
_NPAD = 10240
_ROWS_PER_TILE = _NPAD // _NS
_ZROWS = 128
_NB = 2000


def _sc_degree(edge4, ones_h, zdeg_h):
    mesh = plsc.VectorSubcoreMesh(core_axis_name="c", subcore_axis_name="s")

    @functools.partial(
        pl.kernel,
        out_type=jax.ShapeDtypeStruct((_NC, _NPAD), jnp.float32),
        mesh=mesh,
        compiler_params=pltpu.CompilerParams(use_tc_tiling_on_sc=False),
        scratch_types=[
            pltpu.VMEM((_NCHUNK_DEG, _ACHUNK), jnp.int32),
            pltpu.VMEM((_ACHUNK,), jnp.float32),
            pltpu.VMEM_SHARED((_NPAD,), jnp.float32),
            pltpu.SemaphoreType.DMA,
        ],
    )
    def deg_kernel(edge_h, ones_hbm, z_hbm, out_h, dst_v, ones_v, acc, sem):
        c = lax.axis_index("c")
        s = lax.axis_index("s")
        pltpu.sync_copy(
            edge_h.at[1].at[s].at[pl.ds(c * _NCHUNK_DEG, _NCHUNK_DEG)], dst_v)
        pltpu.sync_copy(ones_hbm, ones_v)
        sl = pl.ds(s * (_NPAD // _NS), _NPAD // _NS)
        pltpu.sync_copy(z_hbm, acc.at[sl])
        plsc.subcore_barrier()

        def fire(j, carry):
            pltpu.async_copy(ones_v, acc.at[dst_v.at[j]], sem, add=True)
            return carry

        lax.fori_loop(0, _NCHUNK_DEG, fire, 0)

        def drain(j, carry):
            pltpu.make_async_copy(ones_v, acc.at[dst_v.at[0]], sem).wait()
            return carry

        lax.fori_loop(0, _NCHUNK_DEG, drain, 0)
        plsc.subcore_barrier()
        pltpu.sync_copy(acc.at[sl], out_h.at[c].at[sl])

    return deg_kernel(edge4, ones_h, zdeg_h)


def _sc_aggregate(y2d, edge4, zrows_h):
    mesh = plsc.VectorSubcoreMesh(core_axis_name="c", subcore_axis_name="s")

    @functools.partial(
        pl.kernel,
        out_type=jax.ShapeDtypeStruct((_NPAD, _D), jnp.float32),
        mesh=mesh,
        compiler_params=pltpu.CompilerParams(use_tc_tiling_on_sc=False),
        scratch_types=[
            pltpu.VMEM((_NCHUNK_AGG, _ACHUNK), jnp.int32),
            pltpu.VMEM((_NCHUNK_AGG, _ACHUNK), jnp.int32),
            pltpu.VMEM((_NBUF, _ACHUNK, _DH), jnp.float32),
            pltpu.VMEM((_ZROWS, _DH), jnp.float32),
            pltpu.VMEM_SHARED((_NPAD, _DH), jnp.float32),
            [pltpu.SemaphoreType.DMA] * _NBUF,
            [pltpu.SemaphoreType.DMA] * _NBUF,
        ],
    )
    def agg_kernel(y_h, edge_h, z_h, out_h,
                   src_v, dst_v, rows_v, zrow_v, acc, sem_g, sem_s):
        c = lax.axis_index("c")
        s = lax.axis_index("s")
        pltpu.sync_copy(edge_h.at[0].at[s], src_v)
        pltpu.sync_copy(edge_h.at[1].at[s], dst_v)
        two = jnp.int32(2)

        def transform(j, carry):
            for t in range(_ACHUNK // 16):
                sl = pl.ds(t * 16, 16)
                src_v[j, sl] = src_v[j, sl] * two + c
            return carry

        lax.fori_loop(0, _NCHUNK_AGG, transform, 0)
        pltpu.sync_copy(z_h, zrow_v)
        for i in range(_ROWS_PER_TILE // _ZROWS):
            pltpu.sync_copy(
                zrow_v, acc.at[pl.ds(s * _ROWS_PER_TILE + i * _ZROWS, _ZROWS)])
        plsc.subcore_barrier()
        yc = y_h

        for b in range(_NBUF):
            pltpu.async_copy(yc.at[src_v.at[b]], rows_v.at[b], sem_g[b])

        def group(g, carry):
            base = g * _NBUF
            for b in range(_NBUF):
                j = base + b
                pltpu.make_async_copy(
                    yc.at[src_v.at[j]], rows_v.at[b], sem_g[b]).wait()
                pltpu.async_copy(
                    rows_v.at[b], acc.at[dst_v.at[j]], sem_s[b], add=True)
            for b in range(_NBUF):
                j = base + b
                pltpu.make_async_copy(
                    rows_v.at[b], acc.at[dst_v.at[j]], sem_s[b]).wait()

                @pl.when(g < _NGROUP - 1)
                def _():
                    pltpu.async_copy(
                        yc.at[src_v.at[base + _NBUF + b]], rows_v.at[b], sem_g[b])
            return carry

        lax.fori_loop(0, _NGROUP, group, 0)
        plsc.subcore_barrier()
        rsl = pl.ds(s * _ROWS_PER_TILE, _ROWS_PER_TILE)
        pltpu.sync_copy(acc.at[rsl], out_h.at[rsl, pl.ds(c * _DH, _DH)])

    return agg_kernel(y2d, edge4, zrows_h)


def _norm_of(d0_ref, d1_ref):
    return lax.rsqrt(jnp.maximum(d0_ref[...] + d1_ref[...], 1.0))


def _tc1_body(d0_ref, d1_ref, x_ref, y_ref):
    y_ref[...] = x_ref[...] * _norm_of(d0_ref, d1_ref)


def _tc1(d0, d1, x):
    return pl.pallas_call(
        _tc1_body,
        out_shape=jax.ShapeDtypeStruct((_N, _D), jnp.float32),
        grid=(_N // _NB,),
        in_specs=[
            pl.BlockSpec((_NB, 1), lambda i: (i, 0)),
            pl.BlockSpec((_NB, 1), lambda i: (i, 0)),
            pl.BlockSpec((_NB, _D), lambda i: (i, 0)),
        ],
        out_specs=pl.BlockSpec((_NB, _D), lambda i: (i, 0)),
    )(d0, d1, x)


def _tc2_body(d0_ref, d1_ref, p_ref, x_ref, w_ref, b_ref, y_ref):
    norm = _norm_of(d0_ref, d1_ref)
    agg = p_ref[...] * norm
    h = jnp.dot(agg, w_ref[...], preferred_element_type=jnp.float32)
    h = jnp.maximum(h + b_ref[...] + x_ref[...], 0.0)
    y_ref[...] = h * norm


def _tc2(d0, d1, p, x, W, b):
    return pl.pallas_call(
        _tc2_body,
        out_shape=jax.ShapeDtypeStruct((_N, _D), jnp.float32),
        grid=(_N // _NB,),
        in_specs=[
            pl.BlockSpec((_NB, 1), lambda i: (i, 0)),
            pl.BlockSpec((_NB, 1), lambda i: (i, 0)),
            pl.BlockSpec((_NB, _D), lambda i: (i, 0)),
            pl.BlockSpec((_NB, _D), lambda i: (i, 0)),
            pl.BlockSpec((_D, _D), lambda i: (0, 0)),
            pl.BlockSpec((1, _D), lambda i: (0, 0)),
        ],
        out_specs=pl.BlockSpec((_NB, _D), lambda i: (i, 0)),
    )(d0, d1, p, x, W, b)


def _tc3_body(d0_ref, d1_ref, p_ref, w_ref, b_ref, y_ref):
    norm = _norm_of(d0_ref, d1_ref)
    agg = p_ref[...] * norm
    h = jnp.dot(agg, w_ref[...], preferred_element_type=jnp.float32)
    y_ref[...] = h + b_ref[...]


def _tc3(d0, d1, p, W, b):
    return pl.pallas_call(
        _tc3_body,
        out_shape=jax.ShapeDtypeStruct((_N, _D), jnp.float32),
        grid=(_N // _NB,),
        in_specs=[
            pl.BlockSpec((_NB, 1), lambda i: (i, 0)),
            pl.BlockSpec((_NB, 1), lambda i: (i, 0)),
            pl.BlockSpec((_NB, _D), lambda i: (i, 0)),
            pl.BlockSpec((_D, _D), lambda i: (0, 0)),
            pl.BlockSpec((1, _D), lambda i: (0, 0)),
        ],
        out_specs=pl.BlockSpec((_NB, _D), lambda i: (i, 0)),
    )(d0, d1, p, W, b)


def kernel(inputs, edge_index, W1, b1, W2, b2):
    x = inputs
    edge_agg = edge_index.reshape(2, _NS, _NCHUNK_AGG, _ACHUNK)
    ones_h = jnp.ones((_ACHUNK,), jnp.float32)
    zdeg_h = jnp.zeros((_NPAD // _NS,), jnp.float32)
    zrows_h = jnp.zeros((_ZROWS, _DH), jnp.float32)

    degp = _sc_degree(edge_agg, ones_h, zdeg_h)
    d0 = degp[0, :_N].reshape(_N, 1)
    d1 = degp[1, :_N].reshape(_N, 1)

    y1 = _tc1(d0, d1, x)
    p = _sc_aggregate(y1.reshape(_N * _NC, _DH), edge_agg, zrows_h)
    y2 = _tc2(d0, d1, p, x, W1, b1.reshape(1, _D))
    q = _sc_aggregate(y2.reshape(_N * _NC, _DH), edge_agg, zrows_h)
    out = _tc3(d0, d1, q, W2, b2.reshape(1, _D))
    return out


# speedup vs baseline: 18.6627x; 1.0190x over previous
import functools

import jax
import jax.numpy as jnp
from jax import lax
from jax.experimental import pallas as pl
from jax.experimental.pallas import tpu as pltpu
from jax.experimental.pallas import tpu_sc as plsc

_N = 10000
_D = 128
_DH = _D // 2
_E = 320000
_NC = 2
_NS = 16
_NW = _NC * _NS
_ACHUNK = 80
_NCHUNK_AGG = _E // _NS // _ACHUNK
_NCHUNK_DEG = _NCHUNK_AGG // _NC
_NBUF = 5
_NGROUP = _NCHUNK_AGG // _NBUF
_NPAD = 10240
_ROWS_PER_TILE = _NPAD // _NS
_ZROWS = 128
_NB = 2000


def _sc_degree(edge4, ones_h, zdeg_h):
    mesh = plsc.VectorSubcoreMesh(core_axis_name="c", subcore_axis_name="s")

    @functools.partial(
        pl.kernel,
        out_type=jax.ShapeDtypeStruct((_NC, _NPAD), jnp.float32),
        mesh=mesh,
        compiler_params=pltpu.CompilerParams(use_tc_tiling_on_sc=False),
        scratch_types=[
            pltpu.VMEM((_NCHUNK_DEG, _ACHUNK), jnp.int32),
            pltpu.VMEM((_ACHUNK,), jnp.float32),
            pltpu.VMEM_SHARED((_NPAD,), jnp.float32),
            pltpu.SemaphoreType.DMA,
        ],
    )
    def deg_kernel(edge_h, ones_hbm, z_hbm, out_h, dst_v, ones_v, acc, sem):
        c = lax.axis_index("c")
        s = lax.axis_index("s")
        pltpu.sync_copy(
            edge_h.at[1].at[s].at[pl.ds(c * _NCHUNK_DEG, _NCHUNK_DEG)], dst_v)
        pltpu.sync_copy(ones_hbm, ones_v)
        sl = pl.ds(s * (_NPAD // _NS), _NPAD // _NS)
        pltpu.sync_copy(z_hbm, acc.at[sl])
        plsc.subcore_barrier()

        def fire(j, carry):
            pltpu.async_copy(ones_v, acc.at[dst_v.at[j]], sem, add=True)
            return carry

        lax.fori_loop(0, _NCHUNK_DEG, fire, 0)

        def drain(j, carry):
            pltpu.make_async_copy(ones_v, acc.at[dst_v.at[0]], sem).wait()
            return carry

        lax.fori_loop(0, _NCHUNK_DEG, drain, 0)
        plsc.subcore_barrier()
        pltpu.sync_copy(acc.at[sl], out_h.at[c].at[sl])

    return deg_kernel(edge4, ones_h, zdeg_h)


def _sc_aggregate(y2d, edge4, zrows_h):
    mesh = plsc.VectorSubcoreMesh(core_axis_name="c", subcore_axis_name="s")

    @functools.partial(
        pl.kernel,
        out_type=jax.ShapeDtypeStruct((_NPAD, _D), jnp.float32),
        mesh=mesh,
        compiler_params=pltpu.CompilerParams(use_tc_tiling_on_sc=False),
        scratch_types=[
            pltpu.VMEM((_NCHUNK_AGG, _ACHUNK), jnp.int32),
            pltpu.VMEM((_NCHUNK_AGG, _ACHUNK), jnp.int32),
            pltpu.VMEM((_NBUF, _ACHUNK, _DH), jnp.float32),
            pltpu.VMEM((_ZROWS, _DH), jnp.float32),
            pltpu.VMEM_SHARED((_NPAD, _DH), jnp.float32),
            [pltpu.SemaphoreType.DMA] * _NBUF,
            [pltpu.SemaphoreType.DMA] * _NBUF,
        ],
    )
    def agg_kernel(y_h, edge_h, z_h, out_h,
                   src_v, dst_v, rows_v, zrow_v, acc, sem_g, sem_s):
        c = lax.axis_index("c")
        s = lax.axis_index("s")
        cp_s = pltpu.async_copy(edge_h.at[0].at[s], src_v, sem_g[0])
        cp_d = pltpu.async_copy(edge_h.at[1].at[s], dst_v, sem_g[1])
        cp_z = pltpu.async_copy(z_h, zrow_v, sem_g[2])
        two = jnp.int32(2)

        def zslice(i):
            return acc.at[pl.ds(s * _ROWS_PER_TILE + i * _ZROWS, _ZROWS)]

        cp_z.wait()
        for i in range(_ROWS_PER_TILE // _ZROWS):
            pltpu.async_copy(zrow_v, zslice(i), sem_s[i])
        cp_s.wait()

        def transform(j, carry):
            for t in range(_ACHUNK // 16):
                sl = pl.ds(t * 16, 16)
                src_v[j, sl] = src_v[j, sl] * two + c
            return carry

        lax.fori_loop(0, _NCHUNK_AGG, transform, 0)
        cp_d.wait()
        for i in range(_ROWS_PER_TILE // _ZROWS):
            pltpu.make_async_copy(zrow_v, zslice(i), sem_s[i]).wait()
        plsc.subcore_barrier()
        yc = y_h

        for b in range(_NBUF):
            pltpu.async_copy(yc.at[src_v.at[b]], rows_v.at[b], sem_g[b])

        def group(g, carry):
            base = g * _NBUF
            for b in range(_NBUF):
                j = base + b
                pltpu.make_async_copy(
                    yc.at[src_v.at[j]], rows_v.at[b], sem_g[b]).wait()
                pltpu.async_copy(
                    rows_v.at[b], acc.at[dst_v.at[j]], sem_s[b], add=True)
            for b in range(_NBUF):
                j = base + b
                pltpu.make_async_copy(
                    rows_v.at[b], acc.at[dst_v.at[j]], sem_s[b]).wait()

                @pl.when(g < _NGROUP - 1)
                def _():
                    pltpu.async_copy(
                        yc.at[src_v.at[base + _NBUF + b]], rows_v.at[b], sem_g[b])
            return carry

        lax.fori_loop(0, _NGROUP, group, 0)
        plsc.subcore_barrier()
        rsl = pl.ds(s * _ROWS_PER_TILE, _ROWS_PER_TILE)
        pltpu.sync_copy(acc.at[rsl], out_h.at[rsl, pl.ds(c * _DH, _DH)])

    return agg_kernel(y2d, edge4, zrows_h)


def _norm_of(d0_ref, d1_ref):
    return lax.rsqrt(jnp.maximum(d0_ref[...] + d1_ref[...], 1.0))


def _tc1_body(d0_ref, d1_ref, x_ref, y_ref):
    y_ref[...] = x_ref[...] * _norm_of(d0_ref, d1_ref)


def _tc1(d0, d1, x):
    return pl.pallas_call(
        _tc1_body,
        out_shape=jax.ShapeDtypeStruct((_N, _D), jnp.float32),
        grid=(_N // _NB,),
        in_specs=[
            pl.BlockSpec((_NB, 1), lambda i: (i, 0)),
            pl.BlockSpec((_NB, 1), lambda i: (i, 0)),
            pl.BlockSpec((_NB, _D), lambda i: (i, 0)),
        ],
        out_specs=pl.BlockSpec((_NB, _D), lambda i: (i, 0)),
    )(d0, d1, x)


def _tc2_body(d0_ref, d1_ref, p_ref, x_ref, w_ref, b_ref, y_ref):
    norm = _norm_of(d0_ref, d1_ref)
    agg = p_ref[...] * norm
    h = jnp.dot(agg, w_ref[...], preferred_element_type=jnp.float32)
    h = jnp.maximum(h + b_ref[...] + x_ref[...], 0.0)
    y_ref[...] = h * norm


def _tc2(d0, d1, p, x, W, b):
    return pl.pallas_call(
        _tc2_body,
        out_shape=jax.ShapeDtypeStruct((_N, _D), jnp.float32),
        grid=(_N // _NB,),
        in_specs=[
            pl.BlockSpec((_NB, 1), lambda i: (i, 0)),
            pl.BlockSpec((_NB, 1), lambda i: (i, 0)),
            pl.BlockSpec((_NB, _D), lambda i: (i, 0)),
            pl.BlockSpec((_NB, _D), lambda i: (i, 0)),
            pl.BlockSpec((_D, _D), lambda i: (0, 0)),
            pl.BlockSpec((1, _D), lambda i: (0, 0)),
        ],
        out_specs=pl.BlockSpec((_NB, _D), lambda i: (i, 0)),
    )(d0, d1, p, x, W, b)


def _tc3_body(d0_ref, d1_ref, p_ref, w_ref, b_ref, y_ref):
    norm = _norm_of(d0_ref, d1_ref)
    agg = p_ref[...] * norm
    h = jnp.dot(agg, w_ref[...], preferred_element_type=jnp.float32)
    y_ref[...] = h + b_ref[...]


def _tc3(d0, d1, p, W, b):
    return pl.pallas_call(
        _tc3_body,
        out_shape=jax.ShapeDtypeStruct((_N, _D), jnp.float32),
        grid=(_N // _NB,),
        in_specs=[
            pl.BlockSpec((_NB, 1), lambda i: (i, 0)),
            pl.BlockSpec((_NB, 1), lambda i: (i, 0)),
            pl.BlockSpec((_NB, _D), lambda i: (i, 0)),
            pl.BlockSpec((_D, _D), lambda i: (0, 0)),
            pl.BlockSpec((1, _D), lambda i: (0, 0)),
        ],
        out_specs=pl.BlockSpec((_NB, _D), lambda i: (i, 0)),
    )(d0, d1, p, W, b)


def kernel(inputs, edge_index, W1, b1, W2, b2):
    x = inputs
    edge_agg = edge_index.reshape(2, _NS, _NCHUNK_AGG, _ACHUNK)
    ones_h = jnp.ones((_ACHUNK,), jnp.float32)
    zdeg_h = jnp.zeros((_NPAD // _NS,), jnp.float32)
    zrows_h = jnp.zeros((_ZROWS, _DH), jnp.float32)

    degp = _sc_degree(edge_agg, ones_h, zdeg_h)
    d0 = degp[0, :_N].reshape(_N, 1)
    d1 = degp[1, :_N].reshape(_N, 1)

    y1 = _tc1(d0, d1, x)
    p = _sc_aggregate(y1.reshape(_N * _NC, _DH), edge_agg, zrows_h)
    y2 = _tc2(d0, d1, p, x, W1, b1.reshape(1, _D))
    q = _sc_aggregate(y2.reshape(_N * _NC, _DH), edge_agg, zrows_h)
    out = _tc3(d0, d1, q, W2, b2.reshape(1, _D))
    return out
